# baseline (speedup 1.0000x reference)
import jax
import jax.numpy as jnp
import numpy as np
from jax.experimental import pallas as pl
from jax.experimental.pallas import tpu as pltpu

N = 1_000_000
NPAD = 1 << 20
ROWS = NPAD // 128
K = 1024
CHUNK = 512

_f = np.float32
INV_SQRT2 = _f(0.7071067690849304)
LOG_2PI = _f(1.83787704)

_P1 = [_f(7.85386146e-05), _f(-0.000801019371), _f(0.00518832775),
       _f(-0.0268538129), _f(0.112835854), _f(-0.37612626), _f(1.12837911)]
_P2 = [_f(0.0232682), _f(-0.138703942), _f(0.368742466), _f(-0.582473278),
       _f(0.621000469), _f(-0.494451523), _f(0.340488), _f(-0.274112701),
       _f(0.563825965)]
_P3 = [_f(-10.477664), _f(12.9772), _f(-7.49551868), _f(2.92101908),
       _f(-1.01526523), _f(0.42184633), _f(-0.282076746), _f(0.564189494)]


def _horner(q, coeffs):
    acc = q * coeffs[0]
    acc = acc + coeffs[1]
    for c in coeffs[2:]:
        acc = acc * q
        acc = acc + c
    return acc


def _erfc_xla(x):
    ax = jnp.abs(x)
    w = x * x
    small = _f(1.0) - ax * _horner(w, _P1)
    q = _f(1.0) / w
    poly = jnp.where(ax < _f(2.0), _horner(q, _P2), _horner(q, _P3))
    big = (jnp.exp(-w) * (_f(1.0) / ax)) * poly
    big = jnp.where(-w < _f(-88.7228394), _f(0.0), big)
    return jnp.where(ax < _f(1.0), small, big)


def _norm_cdf_xla(e):
    h = e * INV_SQRT2
    erfc_h = _erfc_xla(h)
    q = jnp.where(h > _f(0.0), _f(2.0) - erfc_h, erfc_h)
    l = jax.lax.erf(h) + _f(1.0)
    r = jnp.where(jnp.abs(h) < INV_SQRT2, l, q)
    return _f(0.5) * r


def _score_body(loc_ref, scale_ref, out_ref):
    best = jnp.max(loc_ref[...])

    def step(i, _):
        a = loc_ref[pl.ds(i * CHUNK, CHUNK), :]
        b = scale_ref[pl.ds(i * CHUNK, CHUNK), :]
        d = a - best
        e = d / b
        s = _norm_cdf_xla(e)
        x2 = e * e
        pdf = jnp.exp((x2 + LOG_2PI) * _f(-0.5))
        score = d * s + b * pdf
        out_ref[pl.ds(i * CHUNK, CHUNK), :] = score
        return 0

    jax.lax.fori_loop(0, ROWS // CHUNK, step, 0)


def kernel(loc, scale):
    locp = jnp.pad(loc, (0, NPAD - N), constant_values=_f(-1e30)).reshape(ROWS, 128)
    scalep = jnp.pad(scale, (0, NPAD - N), constant_values=_f(1.0)).reshape(ROWS, 128)
    scores = pl.pallas_call(
        _score_body,
        out_shape=jax.ShapeDtypeStruct((ROWS, 128), jnp.float32),
        in_specs=[
            pl.BlockSpec(memory_space=pltpu.VMEM),
            pl.BlockSpec(memory_space=pltpu.VMEM),
        ],
        out_specs=pl.BlockSpec(memory_space=pltpu.VMEM),
    )(locp, scalep)
    score1 = scores.reshape(-1)[:N]
    _, idx = jax.lax.top_k(score1, K)
    return idx


# speedup vs baseline: 3.0601x; 3.0601x over previous
import functools

import jax
import jax.numpy as jnp
import numpy as np
from jax import lax
from jax.experimental import pallas as pl
from jax.experimental.pallas import tpu as pltpu
from jax.experimental.pallas import tpu_sc as plsc

N = 1_000_000
NPAD = 1 << 20
ROWS = NPAD // 128
K = 1024
CHUNK = 512

NW = 16
SHARD = NPAD // NW
CAPW = 128
DTOT = NW * CAPW
DBUF = DTOT
INT_MIN = np.int32(-2147483648)

_f = np.float32
INV_SQRT2 = _f(0.7071067690849304)
LOG_2PI = _f(1.83787704)

_P1 = [_f(7.85386146e-05), _f(-0.000801019371), _f(0.00518832775),
       _f(-0.0268538129), _f(0.112835854), _f(-0.37612626), _f(1.12837911)]
_P2 = [_f(0.0232682), _f(-0.138703942), _f(0.368742466), _f(-0.582473278),
       _f(0.621000469), _f(-0.494451523), _f(0.340488), _f(-0.274112701),
       _f(0.563825965)]
_P3 = [_f(-10.477664), _f(12.9772), _f(-7.49551868), _f(2.92101908),
       _f(-1.01526523), _f(0.42184633), _f(-0.282076746), _f(0.564189494)]


def _horner(q, coeffs):
    acc = q * coeffs[0]
    acc = acc + coeffs[1]
    for c in coeffs[2:]:
        acc = acc * q
        acc = acc + c
    return acc


def _erfc_xla(x):
    ax = jnp.abs(x)
    w = x * x
    small = _f(1.0) - ax * _horner(w, _P1)
    q = _f(1.0) / w
    poly = jnp.where(ax < _f(2.0), _horner(q, _P2), _horner(q, _P3))
    big = (jnp.exp(-w) * (_f(1.0) / ax)) * poly
    big = jnp.where(-w < _f(-88.7228394), _f(0.0), big)
    return jnp.where(ax < _f(1.0), small, big)


def _norm_cdf_xla(e):
    h = e * INV_SQRT2
    erfc_h = _erfc_xla(h)
    qq = jnp.where(h > _f(0.0), _f(2.0) - erfc_h, erfc_h)
    l = lax.erf(h) + _f(1.0)
    r = jnp.where(jnp.abs(h) < INV_SQRT2, l, qq)
    return _f(0.5) * r


def _k1_body(loc_ref, scale_ref, keys_ref, tvec_ref):
    best = jnp.max(loc_ref[...])

    def step(i, _):
        a = loc_ref[pl.ds(i * CHUNK, CHUNK), :]
        b = scale_ref[pl.ds(i * CHUNK, CHUNK), :]
        d = a - best
        e = d / b
        s = _norm_cdf_xla(e)
        pdf = jnp.exp((e * e + LOG_2PI) * _f(-0.5))
        score = d * s + b * pdf
        sb = lax.bitcast_convert_type(score, jnp.int32)
        key = jnp.where(sb < 0, sb ^ jnp.int32(0x7FFFFFFF), sb)
        keys_ref[pl.ds(i * CHUNK, CHUNK), :] = key
        return 0

    lax.fori_loop(0, ROWS // CHUNK, step, 0)

    def bit_step(t, carry):
        prefix, kk = carry
        b = 31 - t
        mask_hi = lax.shift_left(jnp.int32(-1), b)
        cand = prefix | lax.shift_left(jnp.int32(1), b)
        candx = cand ^ INT_MIN
        keys = keys_ref[...]
        match = (keys & mask_hi) == candx
        cnt = jnp.sum(match.astype(jnp.int32))
        take = cnt >= kk
        return (jnp.where(take, cand, prefix), jnp.where(take, kk, kk - cnt))

    prefix, _ = lax.fori_loop(0, 32, bit_step, (jnp.int32(0), jnp.int32(K)))
    tkey = prefix ^ INT_MIN
    tvec_ref[...] = jnp.broadcast_to(tkey, (8, 128))


def _k2_body(keys_hbm, tvec_hbm, dk_hbm, di_hbm, keys_v, ck_v, ci_v, t_v):
    w = lax.axis_index("s")
    base = w * SHARD
    pltpu.sync_copy(keys_hbm.at[pl.ds(base, SHARD)], keys_v)
    pltpu.sync_copy(tvec_hbm, t_v)
    iota16 = lax.iota(jnp.int32, 16)
    tv = t_v[...]

    def pfstep(i, _):
        ck_v[pl.ds(i * 16, 16)] = jnp.full((16,), INT_MIN, jnp.int32)
        ci_v[pl.ds(i * 16, 16)] = jnp.zeros((16,), jnp.int32)
        return 0

    lax.fori_loop(0, (CAPW + 16) // 16, pfstep, 0)

    def step(j, wptr):
        v = keys_v[pl.ds(j * 16, 16)]
        m = v >= tv
        cnt = jnp.sum(m.astype(jnp.int32))

        @pl.when(cnt > 0)
        def _append():
            off = jnp.minimum(wptr, CAPW)
            idxv = jnp.full((16,), base + j * 16, jnp.int32) + iota16
            plsc.store_compressed(ck_v.at[pl.ds(off, 16)], v, mask=m)
            plsc.store_compressed(ci_v.at[pl.ds(off, 16)], idxv, mask=m)

        return wptr + cnt

    lax.fori_loop(0, SHARD // 16, step, jnp.int32(0))

    pltpu.sync_copy(ck_v.at[pl.ds(0, CAPW)], dk_hbm.at[pl.ds(w * CAPW, CAPW)])
    pltpu.sync_copy(ci_v.at[pl.ds(0, CAPW)], di_hbm.at[pl.ds(w * CAPW, CAPW)])


def _k3_body(dk2_ref, di2_ref, dks_ref, dis_ref, out_ref):
    dk2 = dk2_ref[...]
    di2 = di2_ref[...]
    sub = lax.broadcasted_iota(jnp.int32, (8, 128), 0)
    lane = lax.broadcasted_iota(jnp.int32, (8, 128), 1)
    iota2d = sub * 128 + lane

    def step(j, acc):
        kj = dks_ref[j]
        ij = dis_ref[j]
        bet = (dk2 > kj) | ((dk2 == kj) & (di2 < ij))
        rank_j = jnp.sum(bet.astype(jnp.int32))
        return acc + jnp.where(iota2d == rank_j, ij, jnp.int32(0))

    acc = lax.fori_loop(0, DTOT, step, jnp.zeros((8, 128), jnp.int32))
    out_ref[...] = acc


def kernel(loc, scale):
    locp = jnp.pad(loc, (0, NPAD - N), constant_values=_f(-1e30)).reshape(ROWS, 128)
    scalep = jnp.pad(scale, (0, NPAD - N), constant_values=_f(1.0)).reshape(ROWS, 128)

    keys2d, tvec2d = pl.pallas_call(
        _k1_body,
        out_shape=[
            jax.ShapeDtypeStruct((ROWS, 128), jnp.int32),
            jax.ShapeDtypeStruct((8, 128), jnp.int32),
        ],
        in_specs=[
            pl.BlockSpec(memory_space=pltpu.VMEM),
            pl.BlockSpec(memory_space=pltpu.VMEM),
        ],
        out_specs=[
            pl.BlockSpec(memory_space=pltpu.VMEM),
            pl.BlockSpec(memory_space=pltpu.VMEM),
        ],
    )(locp, scalep)

    keys_flat = keys2d.reshape(NPAD)
    tvec16 = tvec2d.reshape(-1)[:16]

    mesh = plsc.VectorSubcoreMesh(core_axis_name="c", subcore_axis_name="s",
                                  num_cores=1)
    k2 = functools.partial(
        pl.kernel,
        mesh=mesh,
        compiler_params=pltpu.CompilerParams(needs_layout_passes=False),
        out_type=[
            jax.ShapeDtypeStruct((DBUF,), jnp.int32),
            jax.ShapeDtypeStruct((DBUF,), jnp.int32),
        ],
        scratch_types=[
            pltpu.VMEM((SHARD,), jnp.int32),
            pltpu.VMEM((CAPW + 16,), jnp.int32),
            pltpu.VMEM((CAPW + 16,), jnp.int32),
            pltpu.VMEM((16,), jnp.int32),
        ],
    )(_k2_body)
    dk, di = k2(keys_flat, tvec16)

    dk_d = dk[:DTOT]
    di_d = di[:DTOT]
    out = pl.pallas_call(
        _k3_body,
        out_shape=jax.ShapeDtypeStruct((8, 128), jnp.int32),
        in_specs=[
            pl.BlockSpec(memory_space=pltpu.VMEM),
            pl.BlockSpec(memory_space=pltpu.VMEM),
            pl.BlockSpec(memory_space=pltpu.SMEM),
            pl.BlockSpec(memory_space=pltpu.SMEM),
        ],
        out_specs=pl.BlockSpec(memory_space=pltpu.VMEM),
    )(dk_d.reshape(16, 128), di_d.reshape(16, 128), dk_d, di_d)

    return out.reshape(K)


# speedup vs baseline: 6.7346x; 2.2008x over previous
import functools

import jax
import jax.numpy as jnp
import numpy as np
from jax import lax
from jax.experimental import pallas as pl
from jax.experimental.pallas import tpu as pltpu
from jax.experimental.pallas import tpu_sc as plsc

N = 1_000_000
NPAD = 1 << 20
ROWS = NPAD // 128
K = 1024
CHUNK = 512

NW = 16
SHARD = NPAD // NW
CAPW = 128
DTOT = NW * CAPW
DBUF = DTOT
INT_MIN = np.int32(-2147483648)

_f = np.float32
INV_SQRT2 = _f(0.7071067690849304)
LOG_2PI = _f(1.83787704)

_P1 = [_f(7.85386146e-05), _f(-0.000801019371), _f(0.00518832775),
       _f(-0.0268538129), _f(0.112835854), _f(-0.37612626), _f(1.12837911)]
_P2 = [_f(0.0232682), _f(-0.138703942), _f(0.368742466), _f(-0.582473278),
       _f(0.621000469), _f(-0.494451523), _f(0.340488), _f(-0.274112701),
       _f(0.563825965)]
_P3 = [_f(-10.477664), _f(12.9772), _f(-7.49551868), _f(2.92101908),
       _f(-1.01526523), _f(0.42184633), _f(-0.282076746), _f(0.564189494)]


def _horner(q, coeffs):
    acc = q * coeffs[0]
    acc = acc + coeffs[1]
    for c in coeffs[2:]:
        acc = acc * q
        acc = acc + c
    return acc


def _erfc_xla(x):
    ax = jnp.abs(x)
    w = x * x
    small = _f(1.0) - ax * _horner(w, _P1)
    q = _f(1.0) / w
    poly = jnp.where(ax < _f(2.0), _horner(q, _P2), _horner(q, _P3))
    big = (jnp.exp(-w) * (_f(1.0) / ax)) * poly
    big = jnp.where(-w < _f(-88.7228394), _f(0.0), big)
    return jnp.where(ax < _f(1.0), small, big)


def _norm_cdf_xla(e):
    h = e * INV_SQRT2
    erfc_h = _erfc_xla(h)
    qq = jnp.where(h > _f(0.0), _f(2.0) - erfc_h, erfc_h)
    l = lax.erf(h) + _f(1.0)
    r = jnp.where(jnp.abs(h) < INV_SQRT2, l, qq)
    return _f(0.5) * r


def _k1_body(loc_ref, scale_ref, keys_ref, tvec_ref):
    best = jnp.max(loc_ref[...])

    def step(i, _):
        a = loc_ref[pl.ds(i * CHUNK, CHUNK), :]
        b = scale_ref[pl.ds(i * CHUNK, CHUNK), :]
        d = a - best
        e = d / b
        s = _norm_cdf_xla(e)
        pdf = jnp.exp((e * e + LOG_2PI) * _f(-0.5))
        score = d * s + b * pdf
        sb = lax.bitcast_convert_type(score, jnp.int32)
        key = jnp.where(sb < 0, sb ^ jnp.int32(0x7FFFFFFF), sb)
        keys_ref[pl.ds(i * CHUNK, CHUNK), :] = key
        return 0

    lax.fori_loop(0, ROWS // CHUNK, step, 0)

    def bit_step(t, carry):
        prefix, kk = carry
        b = 31 - t
        mask_hi = lax.shift_left(jnp.int32(-1), b)
        cand = prefix | lax.shift_left(jnp.int32(1), b)
        candx = cand ^ INT_MIN
        keys = keys_ref[...]
        match = (keys & mask_hi) == candx
        cnt = jnp.sum(match.astype(jnp.int32))
        take = cnt >= kk
        return (jnp.where(take, cand, prefix), jnp.where(take, kk, kk - cnt))

    prefix, _ = lax.fori_loop(0, 32, bit_step, (jnp.int32(0), jnp.int32(K)))
    tkey = prefix ^ INT_MIN
    tvec_ref[...] = jnp.broadcast_to(tkey, (8, 128))


def _k2_body(keys_hbm, tvec_hbm, dk_hbm, di_hbm, keys_v, ck_v, ci_v, t_v):
    w = lax.axis_index("s")
    base = w * SHARD
    pltpu.sync_copy(keys_hbm.at[pl.ds(base, SHARD)], keys_v)
    pltpu.sync_copy(tvec_hbm, t_v)
    iota16 = lax.iota(jnp.int32, 16)
    tv = t_v[...]

    def pfstep(i, _):
        ck_v[pl.ds(i * 16, 16)] = jnp.full((16,), INT_MIN, jnp.int32)
        ci_v[pl.ds(i * 16, 16)] = jnp.zeros((16,), jnp.int32)
        return 0

    lax.fori_loop(0, (CAPW + 16) // 16, pfstep, 0)

    def step(j, wptr):
        v = keys_v[pl.ds(j * 16, 16)]
        m = v >= tv
        cnt = jnp.sum(m.astype(jnp.int32))

        @pl.when(cnt > 0)
        def _append():
            off = jnp.minimum(wptr, CAPW)
            idxv = jnp.full((16,), base + j * 16, jnp.int32) + iota16
            plsc.store_compressed(ck_v.at[pl.ds(off, 16)], v, mask=m)
            plsc.store_compressed(ci_v.at[pl.ds(off, 16)], idxv, mask=m)

        return wptr + cnt

    lax.fori_loop(0, SHARD // 16, step, jnp.int32(0))

    pltpu.sync_copy(ck_v.at[pl.ds(0, CAPW)], dk_hbm.at[pl.ds(w * CAPW, CAPW)])
    pltpu.sync_copy(ci_v.at[pl.ds(0, CAPW)], di_hbm.at[pl.ds(w * CAPW, CAPW)])


def _k3_body(dk2_ref, di2_ref, dks_ref, dis_ref, rank_ref):
    dk2 = dk2_ref[...]
    di2 = di2_ref[...]

    def step(j, acc):
        k0 = dks_ref[j]
        i0 = dis_ref[j]
        b0 = (k0 > dk2) | ((k0 == dk2) & (i0 < di2))
        return acc + b0.astype(jnp.int32)

    acc = lax.fori_loop(0, DTOT, step, jnp.zeros((16, 128), jnp.int32),
                        unroll=8)
    rank_ref[...] = acc


def _k4_body(rank_hbm, di_hbm, out_hbm, rank_v, di_v, il_v):
    w = lax.axis_index("s")
    base = w * CAPW
    pltpu.sync_copy(rank_hbm.at[pl.ds(base, CAPW)], rank_v)
    pltpu.sync_copy(di_hbm.at[pl.ds(base, CAPW)], di_v)
    iota16 = lax.iota(jnp.int32, 16)

    def step(i, _):
        r = rank_v[pl.ds(i * 16, 16)]
        r = jnp.where(r < K, r, K + jnp.full((16,), base + i * 16, jnp.int32) + iota16)
        il_v[pl.ds(i * 16, 16)] = r
        return 0

    lax.fori_loop(0, CAPW // 16, step, 0)
    pltpu.sync_copy(di_v, out_hbm.at[il_v])


def kernel(loc, scale):
    locp = jnp.pad(loc, (0, NPAD - N), constant_values=_f(-1e30)).reshape(ROWS, 128)
    scalep = jnp.pad(scale, (0, NPAD - N), constant_values=_f(1.0)).reshape(ROWS, 128)

    keys2d, tvec2d = pl.pallas_call(
        _k1_body,
        out_shape=[
            jax.ShapeDtypeStruct((ROWS, 128), jnp.int32),
            jax.ShapeDtypeStruct((8, 128), jnp.int32),
        ],
        in_specs=[
            pl.BlockSpec(memory_space=pltpu.VMEM),
            pl.BlockSpec(memory_space=pltpu.VMEM),
        ],
        out_specs=[
            pl.BlockSpec(memory_space=pltpu.VMEM),
            pl.BlockSpec(memory_space=pltpu.VMEM),
        ],
    )(locp, scalep)

    keys_flat = keys2d.reshape(NPAD)
    tvec16 = tvec2d.reshape(-1)[:16]

    mesh = plsc.VectorSubcoreMesh(core_axis_name="c", subcore_axis_name="s",
                                  num_cores=1)
    k2 = functools.partial(
        pl.kernel,
        mesh=mesh,
        compiler_params=pltpu.CompilerParams(needs_layout_passes=False),
        out_type=[
            jax.ShapeDtypeStruct((DBUF,), jnp.int32),
            jax.ShapeDtypeStruct((DBUF,), jnp.int32),
        ],
        scratch_types=[
            pltpu.VMEM((SHARD,), jnp.int32),
            pltpu.VMEM((CAPW + 16,), jnp.int32),
            pltpu.VMEM((CAPW + 16,), jnp.int32),
            pltpu.VMEM((16,), jnp.int32),
        ],
    )(_k2_body)
    dk, di = k2(keys_flat, tvec16)

    dk_d = dk[:DTOT]
    di_d = di[:DTOT]
    ranks = pl.pallas_call(
        _k3_body,
        out_shape=jax.ShapeDtypeStruct((16, 128), jnp.int32),
        in_specs=[
            pl.BlockSpec(memory_space=pltpu.VMEM),
            pl.BlockSpec(memory_space=pltpu.VMEM),
            pl.BlockSpec(memory_space=pltpu.SMEM),
            pl.BlockSpec(memory_space=pltpu.SMEM),
        ],
        out_specs=pl.BlockSpec(memory_space=pltpu.VMEM),
    )(dk_d.reshape(16, 128), di_d.reshape(16, 128), dk_d, di_d)

    k4 = functools.partial(
        pl.kernel,
        mesh=mesh,
        compiler_params=pltpu.CompilerParams(needs_layout_passes=False),
        out_type=jax.ShapeDtypeStruct((K + DTOT,), jnp.int32),
        scratch_types=[
            pltpu.VMEM((CAPW,), jnp.int32),
            pltpu.VMEM((CAPW,), jnp.int32),
            pltpu.VMEM((CAPW,), jnp.int32),
        ],
    )(_k4_body)
    out = k4(ranks.reshape(DTOT), di_d)
    return out[:K]


# speedup vs baseline: 12.8021x; 1.9009x over previous
import functools

import jax
import jax.numpy as jnp
import numpy as np
from jax import lax
from jax.experimental import pallas as pl
from jax.experimental.pallas import tpu as pltpu
from jax.experimental.pallas import tpu_sc as plsc

N = 1_000_000
NPAD = 1 << 20
ROWS = NPAD // 128
K = 1024
CHUNK = 512
POOL_STOP = 256

NW = 32
SHARD = NPAD // NW
CAPW = 96
DTOT = NW * CAPW
DBUF = DTOT
INT_MIN = np.int32(-2147483648)

_f = np.float32
INV_SQRT2 = _f(0.7071067690849304)
LOG_2PI = _f(1.83787704)

_P1 = [_f(7.85386146e-05), _f(-0.000801019371), _f(0.00518832775),
       _f(-0.0268538129), _f(0.112835854), _f(-0.37612626), _f(1.12837911)]
_P2 = [_f(0.0232682), _f(-0.138703942), _f(0.368742466), _f(-0.582473278),
       _f(0.621000469), _f(-0.494451523), _f(0.340488), _f(-0.274112701),
       _f(0.563825965)]
_P3 = [_f(-10.477664), _f(12.9772), _f(-7.49551868), _f(2.92101908),
       _f(-1.01526523), _f(0.42184633), _f(-0.282076746), _f(0.564189494)]


def _horner(q, coeffs):
    acc = q * coeffs[0]
    acc = acc + coeffs[1]
    for c in coeffs[2:]:
        acc = acc * q
        acc = acc + c
    return acc


def _erfc_xla(x):
    ax = jnp.abs(x)
    w = x * x
    small = _f(1.0) - ax * _horner(w, _P1)
    q = _f(1.0) / w
    poly = jnp.where(ax < _f(2.0), _horner(q, _P2), _horner(q, _P3))
    big = (jnp.exp(-w) * (_f(1.0) / ax)) * poly
    big = jnp.where(-w < _f(-88.7228394), _f(0.0), big)
    return jnp.where(ax < _f(1.0), small, big)


def _norm_cdf_xla(e):
    h = e * INV_SQRT2
    erfc_h = _erfc_xla(h)
    qq = jnp.where(h > _f(0.0), _f(2.0) - erfc_h, erfc_h)
    l = lax.erf(h) + _f(1.0)
    r = jnp.where(jnp.abs(h) < INV_SQRT2, l, qq)
    return _f(0.5) * r


def _k1_body(loc_ref, scale_ref, keys_ref, tvec_ref):
    best = jnp.max(loc_ref[...])

    def step(i, _):
        a = loc_ref[pl.ds(i * CHUNK, CHUNK), :]
        b = scale_ref[pl.ds(i * CHUNK, CHUNK), :]
        d = a - best
        e = d / b
        s = _norm_cdf_xla(e)
        pdf = jnp.exp((e * e + LOG_2PI) * _f(-0.5))
        score = d * s + b * pdf
        sb = lax.bitcast_convert_type(score, jnp.int32)
        key = jnp.where(sb < 0, sb ^ jnp.int32(0x7FFFFFFF), sb)
        keys_ref[pl.ds(i * CHUNK, CHUNK), :] = key
        return 0

    lax.fori_loop(0, ROWS // CHUNK, step, 0)

    def bit_cond(carry):
        t, prefix, kk, pool = carry
        return (t < 32) & (pool > POOL_STOP)

    def bit_step(carry):
        t, prefix, kk, pool = carry
        b = 31 - t
        mask_hi = lax.shift_left(jnp.int32(-1), b)
        cand = prefix | lax.shift_left(jnp.int32(1), b)
        candx = cand ^ INT_MIN
        keys = keys_ref[...]
        match = (keys & mask_hi) == candx
        cnt = jnp.sum(match.astype(jnp.int32))
        take = cnt >= kk
        return (t + 1, jnp.where(take, cand, prefix),
                jnp.where(take, kk, kk - cnt),
                jnp.where(take, cnt, pool - cnt))

    _, prefix, _, _ = lax.while_loop(
        bit_cond, bit_step,
        (jnp.int32(0), jnp.int32(0), jnp.int32(K), jnp.int32(NPAD)))
    tkey = prefix ^ INT_MIN
    tvec_ref[...] = jnp.broadcast_to(tkey, (8, 128))


def _k2_body(keys_hbm, tvec_hbm, dk_hbm, di_hbm, keys_v, ck_v, ci_v, t_v):
    w = lax.axis_index("c") * 16 + lax.axis_index("s")
    base = w * SHARD
    pltpu.sync_copy(keys_hbm.at[pl.ds(base, SHARD)], keys_v)
    pltpu.sync_copy(tvec_hbm, t_v)
    iota16 = lax.iota(jnp.int32, 16)
    tv = t_v[...]

    def pfstep(i, _):
        ck_v[pl.ds(i * 16, 16)] = jnp.full((16,), INT_MIN, jnp.int32)
        ci_v[pl.ds(i * 16, 16)] = jnp.zeros((16,), jnp.int32)
        return 0

    lax.fori_loop(0, (CAPW + 16) // 16, pfstep, 0)

    def step(i, wptr):
        vs = [keys_v[pl.ds((i * 8 + t) * 16, 16)] for t in range(8)]
        ms = [v >= tv for v in vs]
        anym = ms[0]
        for t in range(1, 8):
            anym = anym | ms[t]
        acnt = jnp.sum(anym.astype(jnp.int32))

        def append(wp):
            for t in range(8):
                c_t = jnp.sum(ms[t].astype(jnp.int32))
                off = jnp.minimum(wp, CAPW)
                idxv = jnp.full((16,), base + (i * 8 + t) * 16, jnp.int32) + iota16
                plsc.store_compressed(ck_v.at[pl.ds(off, 16)], vs[t], mask=ms[t])
                plsc.store_compressed(ci_v.at[pl.ds(off, 16)], idxv, mask=ms[t])
                wp = wp + c_t
            return wp

        return lax.cond(acnt > 0, append, lambda wp: wp, wptr)

    lax.fori_loop(0, SHARD // 128, step, jnp.int32(0))

    pltpu.sync_copy(ck_v.at[pl.ds(0, CAPW)], dk_hbm.at[pl.ds(w * CAPW, CAPW)])
    pltpu.sync_copy(ci_v.at[pl.ds(0, CAPW)], di_hbm.at[pl.ds(w * CAPW, CAPW)])


def _k3_body(dk2_ref, di2_ref, dks_ref, dis_ref, rank_ref):
    dk2 = dk2_ref[...]
    di2 = di2_ref[...]

    def step(j, acc):
        k0 = dks_ref[j]
        i0 = dis_ref[j]
        b0 = (k0 > dk2) | ((k0 == dk2) & (i0 < di2))
        return acc + b0.astype(jnp.int32)

    acc = lax.fori_loop(0, DTOT, step,
                        jnp.zeros((DTOT // 128, 128), jnp.int32), unroll=8)
    rank_ref[...] = acc


def _k4_body(rank_hbm, di_hbm, out_hbm, rank_v, di_v, il_v):
    w = lax.axis_index("c") * 16 + lax.axis_index("s")
    base = w * CAPW
    pltpu.sync_copy(rank_hbm.at[pl.ds(base, CAPW)], rank_v)
    pltpu.sync_copy(di_hbm.at[pl.ds(base, CAPW)], di_v)
    iota16 = lax.iota(jnp.int32, 16)

    def step(i, _):
        r = rank_v[pl.ds(i * 16, 16)]
        r = jnp.where(r < K, r, K + jnp.full((16,), base + i * 16, jnp.int32) + iota16)
        il_v[pl.ds(i * 16, 16)] = r
        return 0

    lax.fori_loop(0, CAPW // 16, step, 0)
    pltpu.sync_copy(di_v, out_hbm.at[il_v])


def kernel(loc, scale):
    locp = jnp.pad(loc, (0, NPAD - N), constant_values=_f(-1e30)).reshape(ROWS, 128)
    scalep = jnp.pad(scale, (0, NPAD - N), constant_values=_f(1.0)).reshape(ROWS, 128)

    keys2d, tvec2d = pl.pallas_call(
        _k1_body,
        out_shape=[
            jax.ShapeDtypeStruct((ROWS, 128), jnp.int32),
            jax.ShapeDtypeStruct((8, 128), jnp.int32),
        ],
        in_specs=[
            pl.BlockSpec(memory_space=pltpu.VMEM),
            pl.BlockSpec(memory_space=pltpu.VMEM),
        ],
        out_specs=[
            pl.BlockSpec(memory_space=pltpu.VMEM),
            pl.BlockSpec(memory_space=pltpu.VMEM),
        ],
    )(locp, scalep)

    keys_flat = keys2d.reshape(NPAD)
    tvec16 = tvec2d.reshape(-1)[:16]

    mesh = plsc.VectorSubcoreMesh(core_axis_name="c", subcore_axis_name="s",
                                  num_cores=2)
    k2 = functools.partial(
        pl.kernel,
        mesh=mesh,
        compiler_params=pltpu.CompilerParams(needs_layout_passes=False),
        out_type=[
            jax.ShapeDtypeStruct((DBUF,), jnp.int32),
            jax.ShapeDtypeStruct((DBUF,), jnp.int32),
        ],
        scratch_types=[
            pltpu.VMEM((SHARD,), jnp.int32),
            pltpu.VMEM((CAPW + 16,), jnp.int32),
            pltpu.VMEM((CAPW + 16,), jnp.int32),
            pltpu.VMEM((16,), jnp.int32),
        ],
    )(_k2_body)
    dk, di = k2(keys_flat, tvec16)

    dk_d = dk[:DTOT]
    di_d = di[:DTOT]
    ranks = pl.pallas_call(
        _k3_body,
        out_shape=jax.ShapeDtypeStruct((DTOT // 128, 128), jnp.int32),
        in_specs=[
            pl.BlockSpec(memory_space=pltpu.VMEM),
            pl.BlockSpec(memory_space=pltpu.VMEM),
            pl.BlockSpec(memory_space=pltpu.SMEM),
            pl.BlockSpec(memory_space=pltpu.SMEM),
        ],
        out_specs=pl.BlockSpec(memory_space=pltpu.VMEM),
    )(dk_d.reshape(DTOT // 128, 128), di_d.reshape(DTOT // 128, 128), dk_d, di_d)

    k4 = functools.partial(
        pl.kernel,
        mesh=mesh,
        compiler_params=pltpu.CompilerParams(needs_layout_passes=False),
        out_type=jax.ShapeDtypeStruct((K + DTOT,), jnp.int32),
        scratch_types=[
            pltpu.VMEM((CAPW,), jnp.int32),
            pltpu.VMEM((CAPW,), jnp.int32),
            pltpu.VMEM((CAPW,), jnp.int32),
        ],
    )(_k4_body)
    out = k4(ranks.reshape(DTOT), di_d)
    return out[:K]


# speedup vs baseline: 15.1915x; 1.1866x over previous
import functools

import jax
import jax.numpy as jnp
import numpy as np
from jax import lax
from jax.experimental import pallas as pl
from jax.experimental.pallas import tpu as pltpu
from jax.experimental.pallas import tpu_sc as plsc

N = 1_000_000
NPAD = 1 << 20
ROWS = NPAD // 128
K = 1024
CHUNK = 512
POOL_STOP = 256

NW = 32
SHARD = NPAD // NW
CAPW = 96
DTOT = NW * CAPW
DBUF = DTOT
INT_MIN = np.int32(-2147483648)

_f = np.float32
INV_SQRT2 = _f(0.7071067690849304)
LOG_2PI = _f(1.83787704)

_P1 = [_f(7.85386146e-05), _f(-0.000801019371), _f(0.00518832775),
       _f(-0.0268538129), _f(0.112835854), _f(-0.37612626), _f(1.12837911)]
_P2 = [_f(0.0232682), _f(-0.138703942), _f(0.368742466), _f(-0.582473278),
       _f(0.621000469), _f(-0.494451523), _f(0.340488), _f(-0.274112701),
       _f(0.563825965)]
_P3 = [_f(-10.477664), _f(12.9772), _f(-7.49551868), _f(2.92101908),
       _f(-1.01526523), _f(0.42184633), _f(-0.282076746), _f(0.564189494)]


def _horner(q, coeffs):
    acc = q * coeffs[0]
    acc = acc + coeffs[1]
    for c in coeffs[2:]:
        acc = acc * q
        acc = acc + c
    return acc


def _erfc_xla(x):
    ax = jnp.abs(x)
    w = x * x
    small = _f(1.0) - ax * _horner(w, _P1)
    q = _f(1.0) / w
    poly = jnp.where(ax < _f(2.0), _horner(q, _P2), _horner(q, _P3))
    big = (jnp.exp(-w) * (_f(1.0) / ax)) * poly
    big = jnp.where(-w < _f(-88.7228394), _f(0.0), big)
    return jnp.where(ax < _f(1.0), small, big)


def _norm_cdf_xla(e):
    h = e * INV_SQRT2
    erfc_h = _erfc_xla(h)
    qq = jnp.where(h > _f(0.0), _f(2.0) - erfc_h, erfc_h)
    l = lax.erf(h) + _f(1.0)
    r = jnp.where(jnp.abs(h) < INV_SQRT2, l, qq)
    return _f(0.5) * r


def _k1_body(loc_ref, scale_ref, keys_ref, tvec_ref):
    best = jnp.max(loc_ref[...])

    def step(i, _):
        a = loc_ref[pl.ds(i * CHUNK, CHUNK), :]
        b = scale_ref[pl.ds(i * CHUNK, CHUNK), :]
        d = a - best
        e = d / b
        s = _norm_cdf_xla(e)
        pdf = jnp.exp((e * e + LOG_2PI) * _f(-0.5))
        score = d * s + b * pdf
        sb = lax.bitcast_convert_type(score, jnp.int32)
        key = jnp.where(sb < 0, sb ^ jnp.int32(0x7FFFFFFF), sb)
        keys_ref[pl.ds(i * CHUNK, CHUNK), :] = key
        return 0

    lax.fori_loop(0, ROWS // CHUNK, step, 0)

    def bit_cond(carry):
        t, prefix, kk, pool = carry
        return (t < 32) & (pool > POOL_STOP)

    def bit_step(carry):
        t, prefix, kk, pool = carry
        b = 31 - t
        mask_hi = lax.shift_left(jnp.int32(-1), b)
        cand = prefix | lax.shift_left(jnp.int32(1), b)
        candx = cand ^ INT_MIN
        keys = keys_ref[...]
        match = (keys & mask_hi) == candx
        cnt = jnp.sum(match.astype(jnp.int32))
        take = cnt >= kk
        return (t + 1, jnp.where(take, cand, prefix),
                jnp.where(take, kk, kk - cnt),
                jnp.where(take, cnt, pool - cnt))

    _, prefix, _, _ = lax.while_loop(
        bit_cond, bit_step,
        (jnp.int32(0), jnp.int32(0), jnp.int32(K), jnp.int32(NPAD)))
    tkey = prefix ^ INT_MIN
    tvec_ref[...] = jnp.broadcast_to(tkey, (8, 128))


def _k2_body(keys_hbm, tvec_hbm, dk_hbm, di_hbm, keys_v, ck_v, ci_v, t_v):
    w = lax.axis_index("c") * 16 + lax.axis_index("s")
    base = w * SHARD
    pltpu.sync_copy(keys_hbm.at[pl.ds(base, SHARD)], keys_v)
    pltpu.sync_copy(tvec_hbm, t_v)
    iota16 = lax.iota(jnp.int32, 16)
    tv = t_v[...]

    def pfstep(i, _):
        ck_v[pl.ds(i * 16, 16)] = jnp.full((16,), INT_MIN, jnp.int32)
        ci_v[pl.ds(i * 16, 16)] = jnp.zeros((16,), jnp.int32)
        return 0

    lax.fori_loop(0, (CAPW + 16) // 16, pfstep, 0)

    def step(i, wptr):
        vs = [keys_v[pl.ds((i * 8 + t) * 16, 16)] for t in range(8)]
        ms = [v >= tv for v in vs]
        anym = ms[0]
        for t in range(1, 8):
            anym = anym | ms[t]
        acnt = jnp.sum(anym.astype(jnp.int32))

        def append(wp):
            for t in range(8):
                c_t = jnp.sum(ms[t].astype(jnp.int32))
                off = jnp.minimum(wp, CAPW)
                idxv = jnp.full((16,), base + (i * 8 + t) * 16, jnp.int32) + iota16
                plsc.store_compressed(ck_v.at[pl.ds(off, 16)], vs[t], mask=ms[t])
                plsc.store_compressed(ci_v.at[pl.ds(off, 16)], idxv, mask=ms[t])
                wp = wp + c_t
            return wp

        return lax.cond(acnt > 0, append, lambda wp: wp, wptr)

    lax.fori_loop(0, SHARD // 128, step, jnp.int32(0))

    pltpu.sync_copy(ck_v.at[pl.ds(0, CAPW)], dk_hbm.at[pl.ds(w * CAPW, CAPW)])
    pltpu.sync_copy(ci_v.at[pl.ds(0, CAPW)], di_hbm.at[pl.ds(w * CAPW, CAPW)])


def _k3_body(dk2_ref, di2_ref, dks_ref, dis_ref, out_ref):
    dk2 = dk2_ref[...]
    di2 = di2_ref[...]

    def step(j, acc):
        k0 = dks_ref[j]
        i0 = dis_ref[j]
        b0 = (k0 > dk2) | ((k0 == dk2) & (i0 < di2))
        return acc + b0.astype(jnp.int32)

    rank = lax.fori_loop(0, DTOT, step,
                         jnp.zeros((DTOT // 128, 128), jnp.int32), unroll=8)

    iota_p = lax.broadcasted_iota(jnp.int32, (1, K), 1)
    acc_out = jnp.zeros((1, K), jnp.float32)
    for c in range(DTOT // 128):
        rcol = lax.transpose(rank[c:c + 1, :], (1, 0))
        pblk = (rcol == iota_p).astype(jnp.float32)
        drow = di2[c:c + 1, :].astype(jnp.float32)
        acc_out = acc_out + jax.lax.dot_general(
            drow, pblk, (((1,), (0,)), ((), ())),
            precision=lax.Precision.HIGHEST,
            preferred_element_type=jnp.float32)
    out_ref[...] = acc_out.astype(jnp.int32)


def kernel(loc, scale):
    locp = jnp.pad(loc, (0, NPAD - N), constant_values=_f(-1e30)).reshape(ROWS, 128)
    scalep = jnp.pad(scale, (0, NPAD - N), constant_values=_f(1.0)).reshape(ROWS, 128)

    keys2d, tvec2d = pl.pallas_call(
        _k1_body,
        out_shape=[
            jax.ShapeDtypeStruct((ROWS, 128), jnp.int32),
            jax.ShapeDtypeStruct((8, 128), jnp.int32),
        ],
        in_specs=[
            pl.BlockSpec(memory_space=pltpu.VMEM),
            pl.BlockSpec(memory_space=pltpu.VMEM),
        ],
        out_specs=[
            pl.BlockSpec(memory_space=pltpu.VMEM),
            pl.BlockSpec(memory_space=pltpu.VMEM),
        ],
    )(locp, scalep)

    keys_flat = keys2d.reshape(NPAD)
    tvec16 = tvec2d.reshape(-1)[:16]

    mesh = plsc.VectorSubcoreMesh(core_axis_name="c", subcore_axis_name="s",
                                  num_cores=2)
    k2 = functools.partial(
        pl.kernel,
        mesh=mesh,
        compiler_params=pltpu.CompilerParams(needs_layout_passes=False),
        out_type=[
            jax.ShapeDtypeStruct((DBUF,), jnp.int32),
            jax.ShapeDtypeStruct((DBUF,), jnp.int32),
        ],
        scratch_types=[
            pltpu.VMEM((SHARD,), jnp.int32),
            pltpu.VMEM((CAPW + 16,), jnp.int32),
            pltpu.VMEM((CAPW + 16,), jnp.int32),
            pltpu.VMEM((16,), jnp.int32),
        ],
    )(_k2_body)
    dk, di = k2(keys_flat, tvec16)

    dk_d = dk[:DTOT]
    di_d = di[:DTOT]
    out = pl.pallas_call(
        _k3_body,
        out_shape=jax.ShapeDtypeStruct((1, K), jnp.int32),
        in_specs=[
            pl.BlockSpec(memory_space=pltpu.VMEM),
            pl.BlockSpec(memory_space=pltpu.VMEM),
            pl.BlockSpec(memory_space=pltpu.SMEM),
            pl.BlockSpec(memory_space=pltpu.SMEM),
        ],
        out_specs=pl.BlockSpec(memory_space=pltpu.VMEM),
    )(dk_d.reshape(DTOT // 128, 128), di_d.reshape(DTOT // 128, 128), dk_d, di_d)
    return out.reshape(K)
